# Initial kernel scaffold; baseline (speedup 1.0000x reference)
import jax
import jax.numpy as jnp
from jax.experimental import pallas as pl


def kernel(x, cnn_features, tok_table, pos_table, W, b, gamma, beta):
    raise NotImplementedError("write your pallas kernel here")


# speedup vs baseline: 1.4126x; 1.4126x over previous
import functools

import jax
import jax.numpy as jnp
from jax import lax
from jax.experimental import pallas as pl
from jax.experimental.pallas import tpu as pltpu
from jax.experimental.pallas import tpu_sc as plsc

_info = plsc.get_sparse_core_info()
_NC, _NS = _info.num_cores, _info.num_subcores
_NW = _NC * _NS


def _make_sc_gather(V, D, N, chunk=400, nbuf=2):
    assert N % _NW == 0
    rows_per_w = N // _NW
    assert rows_per_w % chunk == 0
    nchunks = rows_per_w // chunk
    mesh = plsc.VectorSubcoreMesh(core_axis_name="c", subcore_axis_name="s")

    @functools.partial(
        pl.kernel,
        mesh=mesh,
        out_type=jax.ShapeDtypeStruct((N, D), jnp.float32),
        scratch_types=[
            pltpu.VMEM((rows_per_w,), jnp.int32),
            pltpu.VMEM((nbuf, chunk, D), jnp.float32),
        ] + [pltpu.SemaphoreType.DMA] * nbuf,
    )
    def gather_kernel(table_hbm, idx_hbm, out_hbm, idx_v, rows_v, *sems):
        wid = lax.axis_index("s") * _NC + lax.axis_index("c")
        base = wid * rows_per_w
        pltpu.sync_copy(idx_hbm.at[pl.ds(base, rows_per_w)], idx_v)
        handles = [None] * nchunks
        for g in range(nbuf):
            handles[g] = pltpu.async_copy(
                table_hbm.at[idx_v.at[pl.ds(g * chunk, chunk)]],
                rows_v.at[g % nbuf], sems[g % nbuf])
        for g in range(nchunks):
            handles[g].wait()
            pltpu.sync_copy(rows_v.at[g % nbuf],
                            out_hbm.at[pl.ds(base + g * chunk, chunk)])
            nxt = g + nbuf
            if nxt < nchunks:
                handles[nxt] = pltpu.async_copy(
                    table_hbm.at[idx_v.at[pl.ds(nxt * chunk, chunk)]],
                    rows_v.at[nxt % nbuf], sems[nxt % nbuf])

    return gather_kernel


def _tc_body(tok_ref, cnn_ref, pos_ref, w_ref, b_ref, gamma_ref, beta_ref,
             out_ref):
    tok = tok_ref[...]
    cnn = cnn_ref[...]
    bb, seq, d = tok.shape
    dense = jnp.dot(cnn.reshape(bb * seq, cnn.shape[-1]), w_ref[...],
                    preferred_element_type=jnp.float32).reshape(bb, seq, d)
    comb = tok + dense + pos_ref[...][None] + b_ref[...][None]
    mean = jnp.mean(comb, axis=-1, keepdims=True)
    cent = comb - mean
    var = jnp.mean(cent * cent, axis=-1, keepdims=True)
    normed = cent * lax.rsqrt(var + 1e-5)
    out_ref[...] = normed * gamma_ref[...][None] + beta_ref[...][None]


def kernel(x, cnn_features, tok_table, pos_table, W, b, gamma, beta):
    B, L = x.shape
    V, D = tok_table.shape
    CD = cnn_features.shape[-1]
    N = B * L

    gathered = _make_sc_gather(V, D, N)(tok_table, x.reshape(N))
    gathered = gathered.reshape(B, L, D)

    BB = 8
    grid = (B // BB,)
    out = pl.pallas_call(
        _tc_body,
        grid=grid,
        in_specs=[
            pl.BlockSpec((BB, L, D), lambda i: (i, 0, 0)),
            pl.BlockSpec((BB, L, CD), lambda i: (i, 0, 0)),
            pl.BlockSpec((L, D), lambda i: (0, 0)),
            pl.BlockSpec((CD, D), lambda i: (0, 0)),
            pl.BlockSpec((1, D), lambda i: (0, 0)),
            pl.BlockSpec((1, D), lambda i: (0, 0)),
            pl.BlockSpec((1, D), lambda i: (0, 0)),
        ],
        out_specs=pl.BlockSpec((BB, L, D), lambda i: (i, 0, 0)),
        out_shape=jax.ShapeDtypeStruct((B, L, D), jnp.float32),
    )(gathered, cnn_features, pos_table, W, b.reshape(1, D),
      gamma.reshape(1, D), beta.reshape(1, D))
    return out
